# baseline (speedup 1.0000x reference)
import jax
import jax.numpy as jnp
from jax import lax
from jax.experimental import pallas as pl
from jax.experimental.pallas import tpu as pltpu
from jax.experimental.pallas import tpu_sc as plsc

B = 16384
KC = 200
H = 64
NUM_CLASSES = 1000000

NC = 2
NS = 16
NW = NC * NS
ROWS_PER_W = B // NW
CB = 128
NB = 8
G1 = 104
G2 = KC - G1
NG = (KC + 15) // 16
KPAD = NG * 16


def _sc_body(h_hbm, cand_hbm, w_hbm, out_hbm, idxc, hc, rows2, outv2, tbuf,
             gsem0, gsem1, gsem2, gsem3, gsem4, gsem5, gsem6, gsem7,
             osem0, osem1, osem2, osem3, osem4, osem5, osem6, osem7):
    wid = lax.axis_index("s") * NC + lax.axis_index("c")
    lanes = lax.iota(jnp.int32, 16)
    gsems = (gsem0, gsem1, gsem2, gsem3, gsem4, gsem5, gsem6, gsem7)
    osems = (osem0, osem1, osem2, osem3, osem4, osem5, osem6, osem7)

    def issue_gathers(i, p):
        pltpu.async_copy(w_hbm.at[idxc.at[i, 0]],
                         rows2.at[p, pl.ds(0, G1)], gsems[p])
        pltpu.async_copy(w_hbm.at[idxc.at[i, 1, pl.ds(0, G2)]],
                         rows2.at[p, pl.ds(G1, G2)], gsems[p])

    def wait_gathers(p):
        pltpu.make_async_copy(w_hbm.at[pl.ds(0, G1)],
                              rows2.at[p, pl.ds(0, G1)], gsems[p]).wait()
        pltpu.make_async_copy(w_hbm.at[pl.ds(0, G2)],
                              rows2.at[p, pl.ds(G1, G2)], gsems[p]).wait()

    def compute_row(i, p):
        hv = [hc[i, pl.ds(c * 16, 16)] for c in range(H // 16)]
        lanes17 = lanes * 17

        def arith(ld):
            a0, a1 = plsc.unpack(ld[0], format=plsc.PackFormat.INTERLEAVED)
            b0, b1 = plsc.unpack(ld[1], format=plsc.PackFormat.INTERLEAVED)
            s = a0 * hv[0]
            t = a1 * hv[1]
            s = s + b0 * hv[2]
            t = t + b1 * hv[3]
            return s + t

        def fused(k0c, qc, k0p, qp):
            prev = None
            acc = [None] * 4
            for kk in range(16):
                cur = ([rows2[p, jnp.minimum(k0c + kk, KC - 1),
                              pl.ds(c * 32, 32)] for c in range(2)]
                       if k0c is not None else None)
                if k0p is not None:
                    gv = plsc.load_gather(tbuf, [jnp.full((16,), qp, jnp.int32),
                                                 lanes17 + kk])
                    a = acc[kk & 3]
                    acc[kk & 3] = gv if a is None else a + gv
                if prev is not None:
                    tbuf[qc, pl.ds((kk - 1) * 17, 16)] = arith(prev)
                prev = cur
            if prev is not None:
                tbuf[qc, pl.ds(15 * 17, 16)] = arith(prev)
            if k0p is not None:
                outv2[p, pl.ds(k0p, 16)] = (acc[0] + acc[1]) + (acc[2] + acc[3])

        fused(0, 0, None, None)

        def g_body(g2, carry3):
            ga = 2 * g2 + 1
            fused(ga * 16, 1, (ga - 1) * 16, 0)
            fused((ga + 1) * 16, 0, ga * 16, 1)
            return carry3

        lax.fori_loop(0, (NG - 1) // 2, g_body, 0)
        fused(None, None, (NG - 1) * 16, 0)

    def chunk_body(ci, carry):
        b0 = wid * ROWS_PER_W + ci * CB
        pltpu.sync_copy(cand_hbm.at[pl.ds(b0, CB)], idxc)
        pltpu.sync_copy(h_hbm.at[pl.ds(b0, CB)], hc)
        for q in range(NB - 1):
            issue_gathers(q, q)

        def ring_body(i4, carry2):
            for q in range(NB):
                i = i4 * NB + q
                wait_gathers(q)
                @pl.when(i < CB - (NB - 1))
                def _():
                    issue_gathers(i + NB - 1, (q + NB - 1) % NB)
                @pl.when(i4 > 0)
                def _():
                    pltpu.make_async_copy(
                        outv2.at[q, pl.ds(0, KC)], out_hbm.at[b0], osems[q]
                    ).wait()
                compute_row(i, q)
                pltpu.async_copy(outv2.at[q, pl.ds(0, KC)],
                                 out_hbm.at[b0 + i], osems[q])
            return carry2

        lax.fori_loop(0, CB // NB, ring_body, 0)
        for q in range(NB):
            pltpu.make_async_copy(outv2.at[q, pl.ds(0, KC)],
                                  out_hbm.at[b0], osems[q]).wait()
        return carry

    lax.fori_loop(0, ROWS_PER_W // CB, chunk_body, 0)


CVT_ROWS = NUM_CLASSES // NW
CVT_CHUNK = 250
CVT_NCH = CVT_ROWS // CVT_CHUNK


def _cvt_body(w_hbm, wb_hbm, fin, fout, isem0, isem1, osem0, osem1):
    wid = lax.axis_index("s") * NC + lax.axis_index("c")
    r0 = wid * CVT_ROWS
    isems = (isem0, isem1)
    osems = (osem0, osem1)

    def issue_in(ci, p):
        pltpu.async_copy(w_hbm.at[pl.ds(r0 + ci * CVT_CHUNK, CVT_CHUNK)],
                         fin.at[p], isems[p])

    def wait_in(p):
        pltpu.make_async_copy(w_hbm.at[pl.ds(0, CVT_CHUNK)], fin.at[p],
                              isems[p]).wait()

    def convert_chunk(ci, p, first):
        def _drain():
            pltpu.make_async_copy(fout.at[p], wb_hbm.at[pl.ds(0, CVT_CHUNK)],
                                  osems[p]).wait()
        if isinstance(first, bool):
            if not first:
                _drain()
        else:
            pl.when(jnp.logical_not(first))(_drain)

        def strip(r8, carry):
            for rr in range(10):
                r = r8 * 10 + rr
                for c in range(2):
                    a = fin[p, r, pl.ds(c * 32, 16)]
                    b = fin[p, r, pl.ds(c * 32 + 16, 16)]
                    fout[p, r, pl.ds(c * 32, 32)] = plsc.pack(
                        a, b, format=plsc.PackFormat.INTERLEAVED)
            return carry

        lax.fori_loop(0, CVT_CHUNK // 10, strip, 0)
        pltpu.async_copy(fout.at[p],
                         wb_hbm.at[pl.ds(r0 + ci * CVT_CHUNK, CVT_CHUNK)],
                         osems[p])

    issue_in(0, 0)

    def pair(c2, carry):
        ca = c2 * 2
        wait_in(0)

        @pl.when(ca + 1 < CVT_NCH)
        def _():
            issue_in(ca + 1, 1)
        convert_chunk(ca, 0, c2 == 0)
        wait_in(1)

        @pl.when(ca + 2 < CVT_NCH)
        def _():
            issue_in(ca + 2, 0)
        convert_chunk(ca + 1, 1, c2 == 0)
        return carry

    lax.fori_loop(0, CVT_NCH // 2, pair, 0)
    wait_in(0)
    convert_chunk(CVT_NCH - 1, 0, False)
    for p in range(2):
        pltpu.make_async_copy(fout.at[p], wb_hbm.at[pl.ds(0, CVT_CHUNK)],
                              osems[p]).wait()


def kernel(h, cand_ids, W):
    cand_ids = cand_ids.astype(jnp.int32)
    cand_a = cand_ids[:, :G1]
    cand_b = jnp.pad(cand_ids[:, G1:], ((0, 0), (0, G1 - G2)))
    cand2 = jnp.stack([cand_a, cand_b], axis=1)

    cvt = pl.kernel(
        _cvt_body,
        out_type=jax.ShapeDtypeStruct((NUM_CLASSES, H), jnp.bfloat16),
        mesh=plsc.VectorSubcoreMesh(core_axis_name="c", subcore_axis_name="s"),
        compiler_params=pltpu.CompilerParams(needs_layout_passes=False,
                                             use_tc_tiling_on_sc=False),
        scratch_types=[
            pltpu.VMEM((2, CVT_CHUNK, H), jnp.float32),
            pltpu.VMEM((2, CVT_CHUNK, H), jnp.bfloat16),
            pltpu.SemaphoreType.DMA,
            pltpu.SemaphoreType.DMA,
            pltpu.SemaphoreType.DMA,
            pltpu.SemaphoreType.DMA,
        ],
    )
    Wb = cvt(W)

    run = pl.kernel(
        _sc_body,
        out_type=jax.ShapeDtypeStruct((B, KC), jnp.float32),
        mesh=plsc.VectorSubcoreMesh(core_axis_name="c", subcore_axis_name="s"),
        compiler_params=pltpu.CompilerParams(needs_layout_passes=False,
                                             use_tc_tiling_on_sc=False),
        scratch_types=[
            pltpu.VMEM((CB, 2, G1), jnp.int32),
            pltpu.VMEM((CB, H), jnp.float32),
            pltpu.VMEM((NB, KC, H), jnp.bfloat16),
            pltpu.VMEM((NB, KPAD), jnp.float32),
            pltpu.VMEM((2, 16 * 17), jnp.float32),
        ] + [pltpu.SemaphoreType.DMA] * 16,
    )
    return run(h, cand2, Wb)


# speedup vs baseline: 1.2470x; 1.2470x over previous
import jax
import jax.numpy as jnp
from jax import lax
from jax.experimental import pallas as pl
from jax.experimental.pallas import tpu as pltpu
from jax.experimental.pallas import tpu_sc as plsc

B = 16384
KC = 200
H = 64
NUM_CLASSES = 1000000

NC = 2
NS = 16
NW = NC * NS
ROWS_PER_W = B // NW
CB = 128
NB = 4
G1 = 104
G2 = KC - G1
NG = (KC + 15) // 16
KPAD = NG * 16


def _sc_body(h_hbm, cand_hbm, w_hbm, out_hbm, idxc, hc, rows2, outv2, tbuf,
             gsem0, gsem1, gsem2, gsem3, osem0, osem1, osem2, osem3):
    wid = lax.axis_index("s") * NC + lax.axis_index("c")
    lanes = lax.iota(jnp.int32, 16)
    gsems = (gsem0, gsem1, gsem2, gsem3)
    osems = (osem0, osem1, osem2, osem3)

    def issue_gathers(i, p):
        pltpu.async_copy(w_hbm.at[idxc.at[i, 0]],
                         rows2.at[p, pl.ds(0, G1)], gsems[p])
        pltpu.async_copy(w_hbm.at[idxc.at[i, 1, pl.ds(0, G2)]],
                         rows2.at[p, pl.ds(G1, G2)], gsems[p])

    def wait_gathers(p):
        pltpu.make_async_copy(w_hbm.at[pl.ds(0, G1)],
                              rows2.at[p, pl.ds(0, G1)], gsems[p]).wait()
        pltpu.make_async_copy(w_hbm.at[pl.ds(0, G2)],
                              rows2.at[p, pl.ds(G1, G2)], gsems[p]).wait()

    def compute_row(i, p):
        hv = [hc[i, pl.ds(c * 16, 16)] for c in range(H // 16)]
        lanes17 = lanes * 17

        def arith(ld):
            s = ld[0] * hv[0]
            t = ld[1] * hv[1]
            s = s + ld[2] * hv[2]
            t = t + ld[3] * hv[3]
            return s + t

        def fused(k0c, qc, k0p, qp):
            prev = None
            acc = [None] * 4
            for kk in range(16):
                cur = ([rows2[p, jnp.minimum(k0c + kk, KC - 1),
                              pl.ds(c * 16, 16)] for c in range(4)]
                       if k0c is not None else None)
                if k0p is not None:
                    gv = plsc.load_gather(tbuf, [jnp.full((16,), qp, jnp.int32),
                                                 lanes17 + kk])
                    a = acc[kk & 3]
                    acc[kk & 3] = gv if a is None else a + gv
                if prev is not None:
                    tbuf[qc, pl.ds((kk - 1) * 17, 16)] = arith(prev)
                prev = cur
            if prev is not None:
                tbuf[qc, pl.ds(15 * 17, 16)] = arith(prev)
            if k0p is not None:
                outv2[p, pl.ds(k0p, 16)] = (acc[0] + acc[1]) + (acc[2] + acc[3])

        fused(0, 0, None, None)

        def g_body(g2, carry3):
            ga = 2 * g2 + 1
            fused(ga * 16, 1, (ga - 1) * 16, 0)
            fused((ga + 1) * 16, 0, ga * 16, 1)
            return carry3

        lax.fori_loop(0, (NG - 1) // 2, g_body, 0)
        fused(None, None, (NG - 1) * 16, 0)

    def chunk_body(ci, carry):
        b0 = wid * ROWS_PER_W + ci * CB
        pltpu.sync_copy(cand_hbm.at[pl.ds(b0, CB)], idxc)
        pltpu.sync_copy(h_hbm.at[pl.ds(b0, CB)], hc)
        for q in range(NB - 1):
            issue_gathers(q, q)

        def ring_body(i4, carry2):
            for q in range(NB):
                i = i4 * NB + q
                wait_gathers(q)
                @pl.when(i < CB - (NB - 1))
                def _():
                    issue_gathers(i + NB - 1, (q + NB - 1) % NB)
                @pl.when(i4 > 0)
                def _():
                    pltpu.make_async_copy(
                        outv2.at[q, pl.ds(0, KC)], out_hbm.at[b0], osems[q]
                    ).wait()
                compute_row(i, q)
                pltpu.async_copy(outv2.at[q, pl.ds(0, KC)],
                                 out_hbm.at[b0 + i], osems[q])
            return carry2

        lax.fori_loop(0, CB // NB, ring_body, 0)
        for q in range(NB):
            pltpu.make_async_copy(outv2.at[q, pl.ds(0, KC)],
                                  out_hbm.at[b0], osems[q]).wait()
        return carry

    lax.fori_loop(0, ROWS_PER_W // CB, chunk_body, 0)


def kernel(h, cand_ids, W):
    cand_ids = cand_ids.astype(jnp.int32)
    cand_a = cand_ids[:, :G1]
    cand_b = jnp.pad(cand_ids[:, G1:], ((0, 0), (0, G1 - G2)))
    cand2 = jnp.stack([cand_a, cand_b], axis=1)

    run = pl.kernel(
        _sc_body,
        out_type=jax.ShapeDtypeStruct((B, KC), jnp.float32),
        mesh=plsc.VectorSubcoreMesh(core_axis_name="c", subcore_axis_name="s"),
        compiler_params=pltpu.CompilerParams(needs_layout_passes=False,
                                             use_tc_tiling_on_sc=False),
        scratch_types=[
            pltpu.VMEM((CB, 2, G1), jnp.int32),
            pltpu.VMEM((CB, H), jnp.float32),
            pltpu.VMEM((NB, KC, H), jnp.float32),
            pltpu.VMEM((NB, KPAD), jnp.float32),
            pltpu.VMEM((2, 16 * 17), jnp.float32),
        ] + [pltpu.SemaphoreType.DMA] * 8,
    )
    return run(h, cand2, W)


# speedup vs baseline: 1.2509x; 1.0031x over previous
import jax
import jax.numpy as jnp
from jax import lax
from jax.experimental import pallas as pl
from jax.experimental.pallas import tpu as pltpu
from jax.experimental.pallas import tpu_sc as plsc

B = 16384
KC = 200
H = 64
NUM_CLASSES = 1000000

NC = 2
NS = 16
NW = NC * NS
ROWS_PER_W = B // NW
CB = 128
NB = 4
G1 = 104
G2 = KC - G1
NG = (KC + 15) // 16
KPAD = NG * 16


def _sc_body(h_hbm, cand_hbm, w_hbm, out_hbm, idxc, hc, rows2, outv2, tbuf,
             gsem0, gsem1, gsem2, gsem3, osem0, osem1, osem2, osem3):
    wid = lax.axis_index("s") * NC + lax.axis_index("c")
    lanes = lax.iota(jnp.int32, 16)
    gsems = (gsem0, gsem1, gsem2, gsem3)
    osems = (osem0, osem1, osem2, osem3)

    def issue_gathers(i, p):
        pltpu.async_copy(w_hbm.at[idxc.at[i, 0, pl.ds(0, 56)]],
                         rows2.at[p, pl.ds(0, 56)], gsems[p])
        pltpu.async_copy(w_hbm.at[idxc.at[i, 0, pl.ds(56, 48)]],
                         rows2.at[p, pl.ds(56, 48)], gsems[p])
        pltpu.async_copy(w_hbm.at[idxc.at[i, 1, pl.ds(0, 48)]],
                         rows2.at[p, pl.ds(G1, 48)], gsems[p])
        pltpu.async_copy(w_hbm.at[idxc.at[i, 1, pl.ds(48, 48)]],
                         rows2.at[p, pl.ds(G1 + 48, 48)], gsems[p])

    def wait_gathers(p):
        pltpu.make_async_copy(w_hbm.at[pl.ds(0, 56)],
                              rows2.at[p, pl.ds(0, 56)], gsems[p]).wait()
        for off in (56, G1, G1 + 48):
            pltpu.make_async_copy(w_hbm.at[pl.ds(0, 48)],
                                  rows2.at[p, pl.ds(off, 48)], gsems[p]).wait()

    def compute_row(i, p):
        hv = [hc[i, pl.ds(c * 16, 16)] for c in range(H // 16)]
        lanes17 = lanes * 17

        def arith(ld):
            s = ld[0] * hv[0]
            t = ld[1] * hv[1]
            s = s + ld[2] * hv[2]
            t = t + ld[3] * hv[3]
            return s + t

        def fused(k0c, qc, k0p, qp):
            prev = None
            acc = [None] * 4
            for kk in range(16):
                cur = ([rows2[p, jnp.minimum(k0c + kk, KC - 1),
                              pl.ds(c * 16, 16)] for c in range(4)]
                       if k0c is not None else None)
                if k0p is not None:
                    gv = plsc.load_gather(tbuf, [jnp.full((16,), qp, jnp.int32),
                                                 lanes17 + kk])
                    a = acc[kk & 3]
                    acc[kk & 3] = gv if a is None else a + gv
                if prev is not None:
                    tbuf[qc, pl.ds((kk - 1) * 17, 16)] = arith(prev)
                prev = cur
            if prev is not None:
                tbuf[qc, pl.ds(15 * 17, 16)] = arith(prev)
            if k0p is not None:
                outv2[p, pl.ds(k0p, 16)] = (acc[0] + acc[1]) + (acc[2] + acc[3])

        fused(0, 0, None, None)

        def g_body(g2, carry3):
            ga = 2 * g2 + 1
            fused(ga * 16, 1, (ga - 1) * 16, 0)
            fused((ga + 1) * 16, 0, ga * 16, 1)
            return carry3

        lax.fori_loop(0, (NG - 1) // 2, g_body, 0)
        fused(None, None, (NG - 1) * 16, 0)

    def chunk_body(ci, carry):
        b0 = wid * ROWS_PER_W + ci * CB
        pltpu.sync_copy(cand_hbm.at[pl.ds(b0, CB)], idxc)
        pltpu.sync_copy(h_hbm.at[pl.ds(b0, CB)], hc)
        for q in range(NB - 1):
            issue_gathers(q, q)

        def ring_body(i4, carry2):
            for q in range(NB):
                i = i4 * NB + q
                wait_gathers(q)
                @pl.when(i < CB - (NB - 1))
                def _():
                    issue_gathers(i + NB - 1, (q + NB - 1) % NB)
                @pl.when(i4 > 0)
                def _():
                    pltpu.make_async_copy(
                        outv2.at[q, pl.ds(0, KC)], out_hbm.at[b0], osems[q]
                    ).wait()
                compute_row(i, q)
                pltpu.async_copy(outv2.at[q, pl.ds(0, KC)],
                                 out_hbm.at[b0 + i], osems[q])
            return carry2

        lax.fori_loop(0, CB // NB, ring_body, 0)
        for q in range(NB):
            pltpu.make_async_copy(outv2.at[q, pl.ds(0, KC)],
                                  out_hbm.at[b0], osems[q]).wait()
        return carry

    lax.fori_loop(0, ROWS_PER_W // CB, chunk_body, 0)


def kernel(h, cand_ids, W):
    cand_ids = cand_ids.astype(jnp.int32)
    cand_a = cand_ids[:, :G1]
    cand_b = jnp.pad(cand_ids[:, G1:], ((0, 0), (0, G1 - G2)))
    cand2 = jnp.stack([cand_a, cand_b], axis=1)

    run = pl.kernel(
        _sc_body,
        out_type=jax.ShapeDtypeStruct((B, KC), jnp.float32),
        mesh=plsc.VectorSubcoreMesh(core_axis_name="c", subcore_axis_name="s"),
        compiler_params=pltpu.CompilerParams(needs_layout_passes=False,
                                             use_tc_tiling_on_sc=False),
        scratch_types=[
            pltpu.VMEM((CB, 2, G1), jnp.int32),
            pltpu.VMEM((CB, H), jnp.float32),
            pltpu.VMEM((NB, KC, H), jnp.float32),
            pltpu.VMEM((NB, KPAD), jnp.float32),
            pltpu.VMEM((2, 16 * 17), jnp.float32),
        ] + [pltpu.SemaphoreType.DMA] * 8,
    )
    return run(h, cand2, W)


# speedup vs baseline: 1.2606x; 1.0077x over previous
import jax
import jax.numpy as jnp
from jax import lax
from jax.experimental import pallas as pl
from jax.experimental.pallas import tpu as pltpu
from jax.experimental.pallas import tpu_sc as plsc

B = 16384
KC = 200
H = 64
NUM_CLASSES = 1000000

NC = 2
NS = 16
NW = NC * NS
ROWS_PER_W = B // NW
CB = 256
NB = 4
G1 = 104
G2 = KC - G1
NG = (KC + 15) // 16
KPAD = NG * 16


def _sc_body(h_hbm, cand_hbm, w_hbm, out_hbm, idxc, hc, rows2, outv2, tbuf,
             gsem0, gsem1, gsem2, gsem3, osem0, osem1, osem2, osem3):
    wid = lax.axis_index("s") * NC + lax.axis_index("c")
    lanes = lax.iota(jnp.int32, 16)
    gsems = (gsem0, gsem1, gsem2, gsem3)
    osems = (osem0, osem1, osem2, osem3)

    def issue_gathers(i, p):
        pltpu.async_copy(w_hbm.at[idxc.at[i, 0, pl.ds(0, 56)]],
                         rows2.at[p, pl.ds(0, 56)], gsems[p])
        pltpu.async_copy(w_hbm.at[idxc.at[i, 0, pl.ds(56, 48)]],
                         rows2.at[p, pl.ds(56, 48)], gsems[p])
        pltpu.async_copy(w_hbm.at[idxc.at[i, 1, pl.ds(0, 48)]],
                         rows2.at[p, pl.ds(G1, 48)], gsems[p])
        pltpu.async_copy(w_hbm.at[idxc.at[i, 1, pl.ds(48, 48)]],
                         rows2.at[p, pl.ds(G1 + 48, 48)], gsems[p])

    def wait_gathers(p):
        pltpu.make_async_copy(w_hbm.at[pl.ds(0, 56)],
                              rows2.at[p, pl.ds(0, 56)], gsems[p]).wait()
        for off in (56, G1, G1 + 48):
            pltpu.make_async_copy(w_hbm.at[pl.ds(0, 48)],
                                  rows2.at[p, pl.ds(off, 48)], gsems[p]).wait()

    def compute_row(i, p):
        hv = [hc[i, pl.ds(c * 16, 16)] for c in range(H // 16)]
        lanes17 = lanes * 17

        def arith(ld):
            s = ld[0] * hv[0]
            t = ld[1] * hv[1]
            s = s + ld[2] * hv[2]
            t = t + ld[3] * hv[3]
            return s + t

        def fused(k0c, qc, k0p, qp):
            prev = None
            acc = [None] * 4
            for kk in range(16):
                cur = ([rows2[p, jnp.minimum(k0c + kk, KC - 1),
                              pl.ds(c * 16, 16)] for c in range(4)]
                       if k0c is not None else None)
                if k0p is not None:
                    gv = plsc.load_gather(tbuf, [jnp.full((16,), qp, jnp.int32),
                                                 lanes17 + kk])
                    a = acc[kk & 3]
                    acc[kk & 3] = gv if a is None else a + gv
                if prev is not None:
                    tbuf[qc, pl.ds((kk - 1) * 17, 16)] = arith(prev)
                prev = cur
            if prev is not None:
                tbuf[qc, pl.ds(15 * 17, 16)] = arith(prev)
            if k0p is not None:
                outv2[p, pl.ds(k0p, 16)] = (acc[0] + acc[1]) + (acc[2] + acc[3])

        fused(0, 0, None, None)

        def g_body(g2, carry3):
            ga = 2 * g2 + 1
            fused(ga * 16, 1, (ga - 1) * 16, 0)
            fused((ga + 1) * 16, 0, ga * 16, 1)
            return carry3

        lax.fori_loop(0, (NG - 1) // 2, g_body, 0)
        fused(None, None, (NG - 1) * 16, 0)

    def chunk_body(ci, carry):
        b0 = wid * ROWS_PER_W + ci * CB
        pltpu.sync_copy(cand_hbm.at[pl.ds(b0, CB)], idxc)
        pltpu.sync_copy(h_hbm.at[pl.ds(b0, CB)], hc)
        for q in range(NB - 1):
            issue_gathers(q, q)

        def ring_body(i4, carry2):
            for q in range(NB):
                i = i4 * NB + q
                wait_gathers(q)
                @pl.when(i < CB - (NB - 1))
                def _():
                    issue_gathers(i + NB - 1, (q + NB - 1) % NB)
                @pl.when(i4 > 0)
                def _():
                    pltpu.make_async_copy(
                        outv2.at[q, pl.ds(0, KC)], out_hbm.at[b0], osems[q]
                    ).wait()
                compute_row(i, q)
                pltpu.async_copy(outv2.at[q, pl.ds(0, KC)],
                                 out_hbm.at[b0 + i], osems[q])
            return carry2

        lax.fori_loop(0, CB // NB, ring_body, 0)
        for q in range(NB):
            pltpu.make_async_copy(outv2.at[q, pl.ds(0, KC)],
                                  out_hbm.at[b0], osems[q]).wait()
        return carry

    lax.fori_loop(0, ROWS_PER_W // CB, chunk_body, 0)


def kernel(h, cand_ids, W):
    cand_ids = cand_ids.astype(jnp.int32)
    cand_a = cand_ids[:, :G1]
    cand_b = jnp.pad(cand_ids[:, G1:], ((0, 0), (0, G1 - G2)))
    cand2 = jnp.stack([cand_a, cand_b], axis=1)

    run = pl.kernel(
        _sc_body,
        out_type=jax.ShapeDtypeStruct((B, KC), jnp.float32),
        mesh=plsc.VectorSubcoreMesh(core_axis_name="c", subcore_axis_name="s"),
        compiler_params=pltpu.CompilerParams(needs_layout_passes=False,
                                             use_tc_tiling_on_sc=False),
        scratch_types=[
            pltpu.VMEM((CB, 2, G1), jnp.int32),
            pltpu.VMEM((CB, H), jnp.float32),
            pltpu.VMEM((NB, KC, H), jnp.float32),
            pltpu.VMEM((NB, KPAD), jnp.float32),
            pltpu.VMEM((2, 16 * 17), jnp.float32),
        ] + [pltpu.SemaphoreType.DMA] * 8,
    )
    return run(h, cand2, W)
